# baseline (speedup 1.0000x reference)
import functools

import jax
import jax.numpy as jnp
from jax import lax
from jax.experimental import pallas as pl
from jax.experimental.pallas import tpu as pltpu
from jax.experimental.pallas import tpu_sc as plsc

_NC = 2
_NS = 16
_NW = _NC * _NS

_B = 16384
_EMB = 32
_BPW = _B // _NW


def _sc_gather(user_id, item_id, user_table, item_table):
    mesh = plsc.VectorSubcoreMesh(
        core_axis_name="c", subcore_axis_name="s", num_cores=_NC,
        num_subcores=_NS)

    n_chunk = _BPW // 128

    @functools.partial(
        pl.kernel,
        out_type=[
            jax.ShapeDtypeStruct((_B, _EMB), jnp.float32),
            jax.ShapeDtypeStruct((_B, _EMB), jnp.float32),
        ],
        mesh=mesh,
        scratch_types=[
            pltpu.VMEM((n_chunk, 128), jnp.int32),
            pltpu.VMEM((n_chunk, 128), jnp.int32),
            pltpu.VMEM((_BPW, _EMB), jnp.float32),
            pltpu.VMEM((_BPW, _EMB), jnp.float32),
            pltpu.SemaphoreType.DMA,
            pltpu.SemaphoreType.DMA,
        ],
        compiler_params=pltpu.CompilerParams(use_tc_tiling_on_sc=False),
    )
    def gather_kernel(uid_hbm, iid_hbm, ut_hbm, it_hbm, u_out, i_out,
                      uidx_v, iidx_v, urows_v, irows_v, usem, isem):
        wid = lax.axis_index("s") * _NC + lax.axis_index("c")
        base = wid * _BPW
        pltpu.sync_copy(uid_hbm.at[wid], uidx_v)
        pltpu.sync_copy(iid_hbm.at[wid], iidx_v)
        for j in range(n_chunk):
            pltpu.async_copy(ut_hbm.at[uidx_v.at[j]],
                             urows_v.at[pl.ds(j * 128, 128)], usem)
            pltpu.async_copy(it_hbm.at[iidx_v.at[j]],
                             irows_v.at[pl.ds(j * 128, 128)], isem)
        for j in range(n_chunk):
            pltpu.make_async_copy(ut_hbm.at[uidx_v.at[j]],
                                  urows_v.at[pl.ds(j * 128, 128)], usem).wait()
            pltpu.make_async_copy(it_hbm.at[iidx_v.at[j]],
                                  irows_v.at[pl.ds(j * 128, 128)], isem).wait()
        pltpu.sync_copy(urows_v, u_out.at[pl.ds(base, _BPW)])
        pltpu.sync_copy(irows_v, i_out.at[pl.ds(base, _BPW)])

    uid3 = user_id.reshape(_NW, n_chunk, 128)
    iid3 = item_id.reshape(_NW, n_chunk, 128)
    return gather_kernel(uid3, iid3, user_table, item_table)


_ROWS_BLK = 2048


def _mlp_body(u_ref, i_ref, w1a_ref, w1b_ref, b1_ref, w2_ref, b2_ref,
              w3_ref, b3_ref, out_ref):
    u = jnp.maximum(u_ref[...], 0.0)
    it = jnp.maximum(i_ref[...], 0.0)
    x = u @ w1a_ref[...] + it @ w1b_ref[...] + b1_ref[...]
    x = jnp.maximum(x, 0.0)
    x = x @ w2_ref[...] + b2_ref[...]
    x = jnp.maximum(x, 0.0)
    out_ref[...] = x @ w3_ref[...] + b3_ref[...]


def _tc_mlp(u_emb, i_emb, W1, b1, W2, b2, W3, b3):
    hid = W1.shape[1]
    w1a = W1[:_EMB]
    w1b = W1[_EMB:]
    b1r = b1.reshape(1, hid)
    b2r = b2.reshape(1, hid)
    b3r = b3.reshape(1, 1)
    n_blk = _B // _ROWS_BLK
    full = lambda i: (0, 0)
    return pl.pallas_call(
        _mlp_body,
        grid=(n_blk,),
        in_specs=[
            pl.BlockSpec((_ROWS_BLK, _EMB), lambda i: (i, 0)),
            pl.BlockSpec((_ROWS_BLK, _EMB), lambda i: (i, 0)),
            pl.BlockSpec(w1a.shape, full),
            pl.BlockSpec(w1b.shape, full),
            pl.BlockSpec(b1r.shape, full),
            pl.BlockSpec(W2.shape, full),
            pl.BlockSpec(b2r.shape, full),
            pl.BlockSpec(W3.shape, full),
            pl.BlockSpec(b3r.shape, full),
        ],
        out_specs=pl.BlockSpec((_ROWS_BLK, 1), lambda i: (i, 0)),
        out_shape=jax.ShapeDtypeStruct((_B, 1), jnp.float32),
    )(u_emb, i_emb, w1a, w1b, b1r, W2, b2r, W3, b3r)


def kernel(user_id, item_id, user_table, item_table, W1, b1, W2, b2, W3, b3):
    u_emb, i_emb = _sc_gather(user_id.astype(jnp.int32),
                              item_id.astype(jnp.int32),
                              user_table, item_table)
    return _tc_mlp(u_emb, i_emb, W1, b1, W2, b2, W3, b3)


# speedup vs baseline: 2.3591x; 2.3591x over previous
import functools

import jax
import jax.numpy as jnp
from jax import lax
from jax.experimental import pallas as pl
from jax.experimental.pallas import tpu as pltpu
from jax.experimental.pallas import tpu_sc as plsc

_NC = 2
_NS = 16
_NW = _NC * _NS

_B = 16384
_EMB = 32
_HID = 7
_BPW = _B // _NW
_NG = _BPW // 16
_NSLOT = 4
_REM_BASE = (1000000 // 128) * 128

_OW1 = 0
_OB1 = _OW1 + 2 * _EMB
_OW2 = _OB1 + 1
_OB2 = _OW2 + _HID
_OW3 = _OB2 + 1
_OB3 = _OW3 + 1
_NWVEC = _OB3 + 1


def _bcast(v, lanev):
    return lax.gather(
        v, lanev.reshape(16, 1),
        lax.GatherDimensionNumbers(offset_dims=(), collapsed_slice_dims=(0,),
                                   start_index_map=(0,)),
        (1,), mode=lax.GatherScatterMode.PROMISE_IN_BOUNDS)


def _pack_weights(W1, b1, W2, b2, W3, b3):
    pad = lambda m: jnp.pad(m, ((0, 0), (0, 16 - m.shape[1])))
    rows = [
        pad(W1.astype(jnp.float32)),
        pad(b1.astype(jnp.float32).reshape(1, _HID)),
        pad(W2.astype(jnp.float32)),
        pad(b2.astype(jnp.float32).reshape(1, _HID)),
        pad(W3.astype(jnp.float32).reshape(1, _HID)),
        pad(b3.astype(jnp.float32).reshape(1, 1)),
    ]
    return jnp.concatenate(rows, axis=0).reshape(-1)


def _fused(uid2, iid2, ut_t, it_t, rem, wall):
    mesh = plsc.VectorSubcoreMesh(core_axis_name="c", subcore_axis_name="s",
                                  num_cores=_NC, num_subcores=_NS)

    @functools.partial(
        pl.kernel,
        out_type=jax.ShapeDtypeStruct((_B,), jnp.float32),
        mesh=mesh,
        scratch_types=[
            pltpu.VMEM((_BPW,), jnp.int32),
            pltpu.VMEM((_BPW,), jnp.int32),
            pltpu.VMEM((_NSLOT, _EMB, 128), jnp.float32),
            pltpu.VMEM((_NSLOT, _EMB, 128), jnp.float32),
            pltpu.VMEM((_NWVEC * 16,), jnp.float32),
            pltpu.VMEM((_BPW,), jnp.float32),
            pltpu.SemaphoreType.DMA,
            pltpu.SemaphoreType.DMA,
        ],
    )
    def kern(uid_hbm, iid_hbm, ut_hbm, it_hbm, rem_hbm, w_hbm, out_hbm,
             uidx_v, iidx_v, uslab_v, islab_v, w_v, out_v, usem, isem):
        wid = lax.axis_index("s") * _NC + lax.axis_index("c")
        base = wid * _BPW
        pltpu.sync_copy(uid_hbm.at[wid], uidx_v)
        pltpu.sync_copy(iid_hbm.at[wid], iidx_v)
        pltpu.sync_copy(w_hbm, w_v)

        c16 = lax.iota(jnp.int32, 16)
        zero = jnp.float32(0.0)
        zv = jnp.zeros((16,), jnp.float32)

        def issue(ru, ri, slot):
            @pl.when(ru < _REM_BASE)
            def _():
                tu = pl.multiple_of((ru >> 7) * 128, 128)
                pltpu.async_copy(ut_hbm.at[:, pl.ds(tu, 128)],
                                 uslab_v.at[slot], usem)

            @pl.when(ru >= _REM_BASE)
            def _():
                pltpu.async_copy(rem_hbm, uslab_v.at[slot], usem)

            @pl.when(ri < _REM_BASE)
            def _():
                ti = pl.multiple_of((ri >> 7) * 128, 128)
                pltpu.async_copy(it_hbm.at[:, pl.ds(ti, 128)],
                                 islab_v.at[slot], isem)

            @pl.when(ri >= _REM_BASE)
            def _():
                pltpu.async_copy(rem_hbm, islab_v.at[slot], isem)

        ru16p = uidx_v[pl.ds(0, 16)]
        ri16p = iidx_v[pl.ds(0, 16)]
        for l in range(_NSLOT - 1):
            issue(ru16p[l], ri16p[l], l)

        def g_body(g, carry):
            ru16 = uidx_v[pl.ds(g * 16, 16)]
            ri16 = iidx_v[pl.ds(g * 16, 16)]
            gn = jnp.minimum(g + 1, _NG - 1)
            ru16n = uidx_v[pl.ds(gn * 16, 16)]
            ri16n = iidx_v[pl.ds(gn * 16, 16)]
            acc_out = zv

            for l in range(16):
                slot = l % _NSLOT
                pltpu.make_async_copy(ut_hbm.at[:, pl.ds(0, 128)],
                                      uslab_v.at[slot], usem).wait()
                pltpu.make_async_copy(it_hbm.at[:, pl.ds(0, 128)],
                                      islab_v.at[slot], isem).wait()

                ru = ru16[l]
                ri = ri16[l]
                lu = jnp.where(ru >= _REM_BASE, ru - _REM_BASE, ru & 127)
                li = jnp.where(ri >= _REM_BASE, (ri - _REM_BASE) + 64,
                               ri & 127)
                uvst = lu & 112
                ivst = li & 112
                ulane = jnp.full((16,), lu & 15, jnp.int32)
                ilane = jnp.full((16,), li & 15, jnp.int32)

                acc = w_v[pl.ds(_OB1 * 16, 16)]
                for d in range(_EMB):
                    uv = uslab_v[slot, d, pl.ds(uvst, 16)]
                    ub = _bcast(uv, ulane)
                    acc = acc + jnp.maximum(ub, zero) * w_v[
                        pl.ds((_OW1 + d) * 16, 16)]
                for d in range(_EMB):
                    iv = islab_v[slot, d, pl.ds(ivst, 16)]
                    ib = _bcast(iv, ilane)
                    acc = acc + jnp.maximum(ib, zero) * w_v[
                        pl.ds((_OW1 + _EMB + d) * 16, 16)]
                x1 = jnp.maximum(acc, zero)

                x2 = w_v[pl.ds(_OB2 * 16, 16)]
                for q in range(_HID):
                    qb = _bcast(x1, jnp.full((16,), q, jnp.int32))
                    x2 = x2 + qb * w_v[pl.ds((_OW2 + q) * 16, 16)]
                x2 = jnp.maximum(x2, zero)

                yv = (x2 * w_v[pl.ds(_OW3 * 16, 16)]
                      + w_v[pl.ds(_OB3 * 16, 16)])
                for k in range(4):
                    yv = yv + _bcast(yv, c16 ^ (1 << k))
                acc_out = acc_out + jnp.where(c16 == l, yv, zv)

                ln = l + _NSLOT - 1
                if ln < 16:
                    run, rin = ru16[ln], ri16[ln]
                else:
                    run, rin = ru16n[ln - 16], ri16n[ln - 16]

                @pl.when((g * 16 + l) + _NSLOT - 1 < _BPW)
                def _():
                    issue(run, rin, (l + _NSLOT - 1) % _NSLOT)

            out_v[pl.ds(g * 16, 16)] = acc_out
            return carry

        lax.fori_loop(0, _NG, g_body, 0)
        pltpu.sync_copy(out_v, out_hbm.at[pl.ds(base, _BPW)])

    return kern(uid2, iid2, ut_t, it_t, rem, wall)


def kernel(user_id, item_id, user_table, item_table, W1, b1, W2, b2, W3, b3):
    uid2 = user_id.astype(jnp.int32).reshape(_NW, _BPW)
    iid2 = item_id.astype(jnp.int32).reshape(_NW, _BPW)
    rem = jnp.concatenate([user_table[_REM_BASE:], item_table[_REM_BASE:]],
                          axis=0).T
    wall = _pack_weights(W1, b1, W2, b2, W3, b3)
    out = _fused(uid2, iid2, user_table.T, item_table.T, rem, wall)
    return out.reshape(_B, 1)


# speedup vs baseline: 2.6406x; 1.1193x over previous
import functools

import jax
import jax.numpy as jnp
from jax import lax
from jax.experimental import pallas as pl
from jax.experimental.pallas import tpu as pltpu
from jax.experimental.pallas import tpu_sc as plsc

_NC = 2
_NS = 16
_NW = _NC * _NS

_B = 16384
_EMB = 32
_HID = 7
_BPW = _B // _NW
_NG = _BPW // 16
_NSLOT = 8
_REM_BASE = (1000000 // 128) * 128

_OW1 = 0
_OB1 = _OW1 + 2 * _EMB
_OW2 = _OB1 + 1
_OB2 = _OW2 + _HID
_OW3 = _OB2 + 1
_OB3 = _OW3 + 1
_NWVEC = _OB3 + 1


def _bcast(v, lanev):
    return lax.gather(
        v, lanev.reshape(16, 1),
        lax.GatherDimensionNumbers(offset_dims=(), collapsed_slice_dims=(0,),
                                   start_index_map=(0,)),
        (1,), mode=lax.GatherScatterMode.PROMISE_IN_BOUNDS)


def _pack_weights(W1, b1, W2, b2, W3, b3):
    pad = lambda m: jnp.pad(m, ((0, 0), (0, 16 - m.shape[1])))
    rows = [
        pad(W1.astype(jnp.float32)),
        pad(b1.astype(jnp.float32).reshape(1, _HID)),
        pad(W2.astype(jnp.float32)),
        pad(b2.astype(jnp.float32).reshape(1, _HID)),
        pad(W3.astype(jnp.float32).reshape(1, _HID)),
        pad(b3.astype(jnp.float32).reshape(1, 1)),
    ]
    return jnp.concatenate(rows, axis=0).reshape(-1)


def _fused(uid2, iid2, ut_t, it_t, rem, wall):
    mesh = plsc.VectorSubcoreMesh(core_axis_name="c", subcore_axis_name="s",
                                  num_cores=_NC, num_subcores=_NS)

    @functools.partial(
        pl.kernel,
        out_type=jax.ShapeDtypeStruct((_B,), jnp.float32),
        mesh=mesh,
        scratch_types=[
            pltpu.VMEM((_BPW,), jnp.int32),
            pltpu.VMEM((_BPW,), jnp.int32),
            pltpu.VMEM((_NSLOT, _EMB, 128), jnp.float32),
            pltpu.VMEM((_NSLOT, _EMB, 128), jnp.float32),
            pltpu.VMEM((_NWVEC * 16,), jnp.float32),
            pltpu.VMEM((_BPW,), jnp.float32),
            pltpu.SemaphoreType.DMA,
            pltpu.SemaphoreType.DMA,
        ],
    )
    def kern(uid_hbm, iid_hbm, ut_hbm, it_hbm, rem_hbm, w_hbm, out_hbm,
             uidx_v, iidx_v, uslab_v, islab_v, w_v, out_v, usem, isem):
        wid = lax.axis_index("s") * _NC + lax.axis_index("c")
        base = wid * _BPW
        pltpu.sync_copy(uid_hbm.at[wid], uidx_v)
        pltpu.sync_copy(iid_hbm.at[wid], iidx_v)
        pltpu.sync_copy(w_hbm, w_v)

        c16 = lax.iota(jnp.int32, 16)
        zero = jnp.float32(0.0)
        zv = jnp.zeros((16,), jnp.float32)

        def issue(ru, ri, slot):
            @pl.when(ru < _REM_BASE)
            def _():
                tu = pl.multiple_of((ru >> 7) * 128, 128)
                pltpu.async_copy(ut_hbm.at[:, pl.ds(tu, 128)],
                                 uslab_v.at[slot], usem)

            @pl.when(ru >= _REM_BASE)
            def _():
                pltpu.async_copy(rem_hbm, uslab_v.at[slot], usem)

            @pl.when(ri < _REM_BASE)
            def _():
                ti = pl.multiple_of((ri >> 7) * 128, 128)
                pltpu.async_copy(it_hbm.at[:, pl.ds(ti, 128)],
                                 islab_v.at[slot], isem)

            @pl.when(ri >= _REM_BASE)
            def _():
                pltpu.async_copy(rem_hbm, islab_v.at[slot], isem)

        ru16p = uidx_v[pl.ds(0, 16)]
        ri16p = iidx_v[pl.ds(0, 16)]
        for l in range(_NSLOT - 1):
            issue(ru16p[l], ri16p[l], l)

        def g_body(g, carry):
            ru16 = uidx_v[pl.ds(g * 16, 16)]
            ri16 = iidx_v[pl.ds(g * 16, 16)]
            gn = jnp.minimum(g + 1, _NG - 1)
            ru16n = uidx_v[pl.ds(gn * 16, 16)]
            ri16n = iidx_v[pl.ds(gn * 16, 16)]
            acc_out = zv

            for l in range(16):
                slot = l % _NSLOT
                pltpu.make_async_copy(ut_hbm.at[:, pl.ds(0, 128)],
                                      uslab_v.at[slot], usem).wait()
                pltpu.make_async_copy(it_hbm.at[:, pl.ds(0, 128)],
                                      islab_v.at[slot], isem).wait()

                ru = ru16[l]
                ri = ri16[l]
                lu = jnp.where(ru >= _REM_BASE, ru - _REM_BASE, ru & 127)
                li = jnp.where(ri >= _REM_BASE, (ri - _REM_BASE) + 64,
                               ri & 127)
                uvst = lu & 112
                ivst = li & 112
                ulane = jnp.full((16,), lu & 15, jnp.int32)
                ilane = jnp.full((16,), li & 15, jnp.int32)

                acc = w_v[pl.ds(_OB1 * 16, 16)]
                for d in range(_EMB):
                    uv = uslab_v[slot, d, pl.ds(uvst, 16)]
                    ub = _bcast(uv, ulane)
                    acc = acc + jnp.maximum(ub, zero) * w_v[
                        pl.ds((_OW1 + d) * 16, 16)]
                for d in range(_EMB):
                    iv = islab_v[slot, d, pl.ds(ivst, 16)]
                    ib = _bcast(iv, ilane)
                    acc = acc + jnp.maximum(ib, zero) * w_v[
                        pl.ds((_OW1 + _EMB + d) * 16, 16)]
                x1 = jnp.maximum(acc, zero)

                x2 = w_v[pl.ds(_OB2 * 16, 16)]
                for q in range(_HID):
                    qb = _bcast(x1, jnp.full((16,), q, jnp.int32))
                    x2 = x2 + qb * w_v[pl.ds((_OW2 + q) * 16, 16)]
                x2 = jnp.maximum(x2, zero)

                yv = (x2 * w_v[pl.ds(_OW3 * 16, 16)]
                      + w_v[pl.ds(_OB3 * 16, 16)])
                for k in range(4):
                    yv = yv + _bcast(yv, c16 ^ (1 << k))
                acc_out = acc_out + jnp.where(c16 == l, yv, zv)

                ln = l + _NSLOT - 1
                if ln < 16:
                    run, rin = ru16[ln], ri16[ln]
                else:
                    run, rin = ru16n[ln - 16], ri16n[ln - 16]

                @pl.when((g * 16 + l) + _NSLOT - 1 < _BPW)
                def _():
                    issue(run, rin, (l + _NSLOT - 1) % _NSLOT)

            out_v[pl.ds(g * 16, 16)] = acc_out
            return carry

        lax.fori_loop(0, _NG, g_body, 0)
        pltpu.sync_copy(out_v, out_hbm.at[pl.ds(base, _BPW)])

    return kern(uid2, iid2, ut_t, it_t, rem, wall)


def kernel(user_id, item_id, user_table, item_table, W1, b1, W2, b2, W3, b3):
    uid2 = user_id.astype(jnp.int32).reshape(_NW, _BPW)
    iid2 = item_id.astype(jnp.int32).reshape(_NW, _BPW)
    rem = jnp.concatenate([user_table[_REM_BASE:], item_table[_REM_BASE:]],
                          axis=0).T
    wall = _pack_weights(W1, b1, W2, b2, W3, b3)
    out = _fused(uid2, iid2, user_table.T, item_table.T, rem, wall)
    return out.reshape(_B, 1)
